# baseline (speedup 1.0000x reference)
import functools

import jax
import jax.numpy as jnp
from jax.experimental import pallas as pl
from jax.experimental.pallas import tpu as pltpu

B, DIM, OUT_DIM, N_Q, BINS = 4096, 512, 1536, 12, 1024
BLOCK_B = 256
MASK_LANES = 128


def _c2_kernel(cb_ref, c2_ref):
    ones_row = jnp.ones((1, DIM), dtype=jnp.float32)
    for q in range(N_Q):
        cb = cb_ref[q]
        c2_ref[q] = jax.lax.dot_general(
            ones_row, cb * cb, (((1,), (1,)), ((), ())),
            precision=jax.lax.Precision.HIGHEST,
            preferred_element_type=jnp.float32)


def _rvq_proj_kernel(embed_ref, cb_ref, c2_ref, w_ref, b_ref,
                     empty_ref, out_ref, mask_ref):
    i = pl.program_id(0)
    residual = embed_ref[...]
    quant_sum = jnp.zeros_like(residual)
    r2 = jnp.sum(residual * residual, axis=1, keepdims=True)
    for q in range(N_Q):
        cb = cb_ref[q]
        cross_m2 = jax.lax.dot_general(
            -2.0 * residual, cb, (((1,), (1,)), ((), ())),
            preferred_element_type=jnp.float32)
        dist = (r2 + cross_m2) + c2_ref[q]
        idx = jnp.argmin(dist, axis=1)
        lane = jax.lax.broadcasted_iota(jnp.int32, dist.shape, 1)
        onehot = (lane == idx[:, None]).astype(jnp.bfloat16)
        cb_hi = cb.astype(jnp.bfloat16)
        rem = cb - cb_hi.astype(jnp.float32)
        cb_mid = rem.astype(jnp.bfloat16)
        cb_lo = (rem - cb_mid.astype(jnp.float32)).astype(jnp.bfloat16)
        quant = (jax.lax.dot_general(
            onehot, cb_hi, (((1,), (0,)), ((), ())),
            preferred_element_type=jnp.float32)
            + jax.lax.dot_general(
            onehot, cb_mid, (((1,), (0,)), ((), ())),
            preferred_element_type=jnp.float32)
            + jax.lax.dot_general(
            onehot, cb_lo, (((1,), (0,)), ((), ())),
            preferred_element_type=jnp.float32))
        quant_sum = quant_sum + quant
        if q < N_Q - 1:
            residual = residual - quant
            r2 = jnp.sum(residual * residual, axis=1, keepdims=True)
    out = jax.lax.dot_general(
        quant_sum, w_ref[...], (((1,), (0,)), ((), ())),
        preferred_element_type=jnp.float32) + b_ref[...]
    rows = i * BLOCK_B + jax.lax.broadcasted_iota(
        jnp.int32, (BLOCK_B, 1), 0)
    hit = jnp.any(rows == empty_ref[...], axis=1, keepdims=True)
    mask = jnp.where(hit, 0.0, 1.0).astype(jnp.float32)
    out_ref[...] = out * mask
    mask_ref[...] = mask


@jax.jit
def kernel(embed, codebooks, W, b, empty_idx):
    c2 = pl.pallas_call(
        _c2_kernel,
        in_specs=[pl.BlockSpec((N_Q, BINS, DIM), lambda: (0, 0, 0))],
        out_specs=pl.BlockSpec((N_Q, 1, BINS), lambda: (0, 0, 0)),
        out_shape=jax.ShapeDtypeStruct((N_Q, 1, BINS), jnp.float32),
    )(codebooks)
    n_blocks = B // BLOCK_B
    out, mask = pl.pallas_call(
        _rvq_proj_kernel,
        grid=(n_blocks,),
        in_specs=[
            pl.BlockSpec((BLOCK_B, DIM), lambda i: (i, 0)),
            pl.BlockSpec((N_Q, BINS, DIM), lambda i: (0, 0, 0)),
            pl.BlockSpec((N_Q, 1, BINS), lambda i: (0, 0, 0)),
            pl.BlockSpec((DIM, OUT_DIM), lambda i: (0, 0)),
            pl.BlockSpec((1, OUT_DIM), lambda i: (0, 0)),
            pl.BlockSpec((1, empty_idx.shape[0]), lambda i: (0, 0)),
        ],
        out_specs=[
            pl.BlockSpec((BLOCK_B, OUT_DIM), lambda i: (i, 0)),
            pl.BlockSpec((BLOCK_B, 1), lambda i: (i, 0)),
        ],
        out_shape=[
            jax.ShapeDtypeStruct((B, OUT_DIM), jnp.float32),
            jax.ShapeDtypeStruct((B, 1), jnp.float32),
        ],
        compiler_params=pltpu.CompilerParams(
            dimension_semantics=("parallel",)),
    )(embed, codebooks, c2, W, b.reshape(1, OUT_DIM),
      empty_idx.reshape(1, -1))
    return out.reshape(B, 1, OUT_DIM), mask


# speedup vs baseline: 1.0872x; 1.0872x over previous
import functools

import jax
import jax.numpy as jnp
from jax.experimental import pallas as pl
from jax.experimental.pallas import tpu as pltpu

B, DIM, OUT_DIM, N_Q, BINS = 4096, 512, 1536, 12, 1024
BLOCK_B = 512
MASK_LANES = 128


def _c2_kernel(cb_ref, c2_ref):
    ones_row = jnp.ones((1, DIM), dtype=jnp.bfloat16)
    for q in range(N_Q):
        cb = cb_ref[q]
        sq = cb * cb
        s_hi = sq.astype(jnp.bfloat16)
        rem = sq - s_hi.astype(jnp.float32)
        s_mid = rem.astype(jnp.bfloat16)
        s_lo = (rem - s_mid.astype(jnp.float32)).astype(jnp.bfloat16)
        acc = jnp.zeros((1, BINS), dtype=jnp.float32)
        for part in (s_lo, s_mid, s_hi):
            acc = acc + jax.lax.dot_general(
                ones_row, part, (((1,), (1,)), ((), ())),
                preferred_element_type=jnp.float32)
        c2_ref[q] = acc


def _rvq_proj_kernel(embed_ref, cb_ref, c2_ref, w_ref, b_ref,
                     empty_ref, out_ref, mask_ref):
    i = pl.program_id(0)
    residual = embed_ref[...]
    quant_sum = jnp.zeros_like(residual)
    r2 = jnp.sum(residual * residual, axis=1, keepdims=True)
    for q in range(N_Q):
        cb = cb_ref[q]
        cross_m2 = jax.lax.dot_general(
            -2.0 * residual, cb, (((1,), (1,)), ((), ())),
            preferred_element_type=jnp.float32)
        dist = (r2 + cross_m2) + c2_ref[q]
        idx = jnp.argmin(dist, axis=1)
        lane = jax.lax.broadcasted_iota(jnp.int32, dist.shape, 1)
        onehot = (lane == idx[:, None]).astype(jnp.bfloat16)
        cb_hi = cb.astype(jnp.bfloat16)
        rem = cb - cb_hi.astype(jnp.float32)
        cb_mid = rem.astype(jnp.bfloat16)
        cb_lo = (rem - cb_mid.astype(jnp.float32)).astype(jnp.bfloat16)
        quant = (jax.lax.dot_general(
            onehot, cb_hi, (((1,), (0,)), ((), ())),
            preferred_element_type=jnp.float32)
            + jax.lax.dot_general(
            onehot, cb_mid, (((1,), (0,)), ((), ())),
            preferred_element_type=jnp.float32)
            + jax.lax.dot_general(
            onehot, cb_lo, (((1,), (0,)), ((), ())),
            preferred_element_type=jnp.float32))
        quant_sum = quant_sum + quant
        if q < N_Q - 1:
            residual = residual - quant
            r2 = jnp.sum(residual * residual, axis=1, keepdims=True)
    out = jax.lax.dot_general(
        quant_sum, w_ref[...], (((1,), (0,)), ((), ())),
        preferred_element_type=jnp.float32) + b_ref[...]
    rows = i * BLOCK_B + jax.lax.broadcasted_iota(
        jnp.int32, (BLOCK_B, 1), 0)
    hit = jnp.any(rows == empty_ref[...], axis=1, keepdims=True)
    mask = jnp.where(hit, 0.0, 1.0).astype(jnp.float32)
    out_ref[...] = out * mask
    mask_ref[...] = mask


@jax.jit
def kernel(embed, codebooks, W, b, empty_idx):
    c2 = pl.pallas_call(
        _c2_kernel,
        in_specs=[pl.BlockSpec((N_Q, BINS, DIM), lambda: (0, 0, 0))],
        out_specs=pl.BlockSpec((N_Q, 1, BINS), lambda: (0, 0, 0)),
        out_shape=jax.ShapeDtypeStruct((N_Q, 1, BINS), jnp.float32),
    )(codebooks)
    n_blocks = B // BLOCK_B
    out, mask = pl.pallas_call(
        _rvq_proj_kernel,
        grid=(n_blocks,),
        in_specs=[
            pl.BlockSpec((BLOCK_B, DIM), lambda i: (i, 0)),
            pl.BlockSpec((N_Q, BINS, DIM), lambda i: (0, 0, 0)),
            pl.BlockSpec((N_Q, 1, BINS), lambda i: (0, 0, 0)),
            pl.BlockSpec((DIM, OUT_DIM), lambda i: (0, 0)),
            pl.BlockSpec((1, OUT_DIM), lambda i: (0, 0)),
            pl.BlockSpec((1, empty_idx.shape[0]), lambda i: (0, 0)),
        ],
        out_specs=[
            pl.BlockSpec((BLOCK_B, OUT_DIM), lambda i: (i, 0)),
            pl.BlockSpec((BLOCK_B, 1), lambda i: (i, 0)),
        ],
        out_shape=[
            jax.ShapeDtypeStruct((B, OUT_DIM), jnp.float32),
            jax.ShapeDtypeStruct((B, 1), jnp.float32),
        ],
        compiler_params=pltpu.CompilerParams(
            dimension_semantics=("parallel",)),
    )(embed, codebooks, c2, W, b.reshape(1, OUT_DIM),
      empty_idx.reshape(1, -1))
    return out.reshape(B, 1, OUT_DIM), mask
